# baseline (speedup 1.0000x reference)
import functools

import jax
import jax.numpy as jnp
from jax.experimental import pallas as pl
from jax.experimental.pallas import tpu as pltpu


def _branch_body(adj_ref, meth_ref, w1_ref, fc1w_ref, w2_ref,
                 b1_ref, b2_ref, fc1b_ref, pool_ref, g_ref, *, n, tm):
    x = meth_ref[0]
    h0 = (
        jnp.dot(x, w1_ref[...], preferred_element_type=jnp.float32)
        .astype(jnp.bfloat16)
    )
    init = (
        jnp.dot(x, fc1w_ref[...], preferred_element_type=jnp.float32)
        + fc1b_ref[...]
    )
    b1 = b1_ref[...]
    w2 = w2_ref[...].astype(jnp.bfloat16)

    starts = list(range(0, n, tm))
    for t0 in starts:
        rows = min(tm, n - t0)
        a_t = adj_ref[0, t0:t0 + rows, :]
        h1_t = jnp.maximum(
            jnp.dot(a_t, h0, preferred_element_type=jnp.float32) + b1,
            0.0,
        ).astype(jnp.bfloat16)
        g_ref[t0:t0 + rows, :] = jnp.dot(
            h1_t, w2, preferred_element_type=jnp.float32
        ).astype(jnp.bfloat16)

    b2 = b2_ref[...]
    g = g_ref[...]
    m = None
    for t0 in starts:
        rows = min(tm, n - t0)
        a_t = adj_ref[0, t0:t0 + rows, :]
        o = (
            jnp.dot(a_t, g, preferred_element_type=jnp.float32)
            + init[t0:t0 + rows, :]
            + b2
        )
        tmax = jnp.max(o, axis=0, keepdims=True)
        m = tmax if m is None else jnp.maximum(m, tmax)
    pool_ref[0] = m


def _branch(adj_bf16, meth, w1, b1, w2, b2, fc1w, fc1b):
    bsz, n, f = meth.shape
    h = w1.shape[1]
    c = w2.shape[1]
    tm = 528

    pool = pl.pallas_call(
        functools.partial(_branch_body, n=n, tm=tm),
        grid=(bsz,),
        in_specs=[
            pl.BlockSpec((1, n, n), lambda b: (b, 0, 0)),
            pl.BlockSpec((1, n, f), lambda b: (b, 0, 0)),
            pl.BlockSpec((f, h), lambda b: (0, 0)),
            pl.BlockSpec((f, c), lambda b: (0, 0)),
            pl.BlockSpec((h, c), lambda b: (0, 0)),
            pl.BlockSpec((1, h), lambda b: (0, 0)),
            pl.BlockSpec((1, c), lambda b: (0, 0)),
            pl.BlockSpec((1, c), lambda b: (0, 0)),
        ],
        out_specs=pl.BlockSpec((1, 1, c), lambda b: (b, 0, 0)),
        out_shape=jax.ShapeDtypeStruct((bsz, 1, c), jnp.float32),
        scratch_shapes=[
            pltpu.VMEM((n, c), jnp.bfloat16),
        ],
        compiler_params=pltpu.CompilerParams(
            dimension_semantics=("arbitrary",),
            vmem_limit_bytes=64 * 1024 * 1024,
        ),
    )(adj_bf16, meth, w1, fc1w, w2, b1, b2, fc1b)

    return pool.reshape(bsz, c)


def _mlp_body(sp_ref, vp_ref, w2a_ref, w2b_ref, b2_ref, w3_ref, b3_ref,
              w4_ref, b4_ref, w5_ref, b5_ref, out_ref):
    d = jnp.maximum(
        jnp.dot(sp_ref[...], w2a_ref[...], preferred_element_type=jnp.float32)
        + jnp.dot(vp_ref[...], w2b_ref[...], preferred_element_type=jnp.float32)
        + b2_ref[...],
        0.0,
    )
    d = jnp.maximum(
        jnp.dot(d, w3_ref[...], preferred_element_type=jnp.float32)
        + b3_ref[...],
        0.0,
    )
    d = jnp.maximum(
        jnp.dot(d, w4_ref[...], preferred_element_type=jnp.float32)
        + b4_ref[...],
        0.0,
    )
    out_ref[...] = (
        jnp.sum(d * w5_ref[...].T, axis=1, keepdims=True) + b5_ref[...]
    )


def kernel(solute_adj, solute_meth, solvent_meth, solvent_adj_meth,
           conv1_W, conv1_b, conv2_W, conv2_b,
           fc1_W, fc1_b, fc2_W, fc2_b, fc3_W, fc3_b,
           fc4_W, fc4_b, fc5_W, fc5_b):
    b1 = conv1_b.reshape(1, -1)
    b2 = conv2_b.reshape(1, -1)
    fb1 = fc1_b.reshape(1, -1)
    nclass = fc1_W.shape[1]

    su_adj = solute_adj.astype(jnp.bfloat16)
    sv_adj = solvent_adj_meth.astype(jnp.bfloat16)

    su_pool = _branch(su_adj, solute_meth, conv1_W, b1, conv2_W, b2,
                      fc1_W, fb1)
    sv_pool = _branch(sv_adj, solvent_meth, conv1_W, b1, conv2_W,
                      b2, fc1_W, fb1)

    bsz = su_pool.shape[0]
    out = pl.pallas_call(
        _mlp_body,
        out_shape=jax.ShapeDtypeStruct((bsz, 1), jnp.float32),
    )(su_pool, sv_pool,
      fc2_W[:nclass], fc2_W[nclass:], fc2_b.reshape(1, -1),
      fc3_W, fc3_b.reshape(1, -1),
      fc4_W, fc4_b.reshape(1, -1),
      fc5_W, fc5_b.reshape(1, -1))
    return out
